# baseline (speedup 1.0000x reference)
import functools

import jax
import jax.numpy as jnp
from jax import lax
from jax.experimental import pallas as pl
from jax.experimental.pallas import tpu as pltpu
from jax.experimental.pallas import tpu_sc as plsc

VOCAB = 1000000
EMBD = 32
B = 4096
L = 200

NC = 2
NS = 16
NW = NC * NS

BT = B // 128
LB = L // 8
TPAD = 131
KU = 5
NGRP = L // KU


def _make_formatter():
    mesh = plsc.VectorSubcoreMesh(core_axis_name="c", subcore_axis_name="s")

    @functools.partial(
        pl.kernel,
        out_type=jax.ShapeDtypeStruct((B * L,), jnp.int32),
        mesh=mesh,
        compiler_params=pltpu.CompilerParams(use_tc_tiling_on_sc=True),
        scratch_types=[
            pltpu.VMEM((LB, 8, 128), jnp.int32),
            pltpu.VMEM((LB * 1024,), jnp.int32),
            pltpu.SemaphoreType.DMA,
        ],
    )
    def fmt_kernel(xt_hbm, out_hbm, buf, flat, sem):
        wid = lax.axis_index("s") * NC + lax.axis_index("c")
        for lb in range(LB):
            pltpu.async_copy(
                xt_hbm.at[pl.ds(lb * 8, 8), pl.ds(wid * 128, 128)],
                buf.at[lb],
                sem,
            )
        for lb in range(LB):
            pltpu.make_async_copy(
                xt_hbm.at[pl.ds(lb * 8, 8), pl.ds(wid * 128, 128)],
                buf.at[lb],
                sem,
            ).wait()

        @pl.loop(0, LB)
        def _flatten(lb):
            for r in range(8):
                for c in range(8):
                    flat[pl.ds(lb * 1024 + r * 128 + c * 16, 16)] = (
                        buf[lb, r, pl.ds(c * 16, 16)]
                    )

        @pl.loop(0, LB)
        def _out(lb):
            pltpu.sync_copy(
                flat.at[pl.ds(lb * 1024, 1024)],
                out_hbm.at[pl.ds((lb * BT + wid) * 1024, 1024)],
            )

    return fmt_kernel


def _make_kernel():
    mesh = plsc.VectorSubcoreMesh(core_axis_name="c", subcore_axis_name="s")

    @functools.partial(
        pl.kernel,
        out_type=jax.ShapeDtypeStruct((L, EMBD // 8, BT, 8, 128), jnp.float32),
        mesh=mesh,
        compiler_params=pltpu.CompilerParams(
            use_tc_tiling_on_sc=False, needs_layout_passes=False
        ),
        scratch_types=[
            pltpu.VMEM((LB, 1024), jnp.int32),
            pltpu.VMEM((KU * 128, EMBD), jnp.float32),
            pltpu.VMEM((KU * 128, EMBD), jnp.float32),
            pltpu.VMEM((KU, EMBD // 8, 8, TPAD), jnp.float32),
            pltpu.VMEM((KU, EMBD // 8, 8, TPAD), jnp.float32),
            pltpu.SemaphoreType.DMA,
            pltpu.SemaphoreType.DMA,
            pltpu.SemaphoreType.DMA,
            pltpu.SemaphoreType.DMA,
        ],
    )
    def emb_kernel(table_hbm, x1_hbm, out_hbm,
                   idx_v, gb0, gb1, tb0, tb1, g0, g1, s0, s1):
        gbuf = (gb0, gb1)
        tbuf = (tb0, tb1)
        gsem = (g0, g1)
        ssem = (s0, s1)
        wid = lax.axis_index("s") * NC + lax.axis_index("c")
        lanes = lax.iota(jnp.int32, 16)

        for lb in range(LB):
            pltpu.async_copy(
                x1_hbm.at[pl.ds((lb * BT + wid) * 1024, 1024)],
                idx_v.at[lb],
                g0,
            )
        for lb in range(LB):
            pltpu.make_async_copy(
                x1_hbm.at[pl.ds((lb * BT + wid) * 1024, 1024)],
                idx_v.at[lb],
                g0,
            ).wait()

        def gather_start(g, s):
            for u in range(KU):
                l = g * KU + u
                pltpu.async_copy(
                    table_hbm.at[idx_v.at[l // 8, pl.ds((l % 8) * 128, 128)]],
                    gbuf[s].at[pl.ds(u * 128, 128)],
                    gsem[s],
                )

        def gather_wait(g, s):
            for u in range(KU):
                l = g * KU + u
                pltpu.make_async_copy(
                    table_hbm.at[idx_v.at[l // 8, pl.ds((l % 8) * 128, 128)]],
                    gbuf[s].at[pl.ds(u * 128, 128)],
                    gsem[s],
                ).wait()

        def store_start(g, s):
            pltpu.async_copy(
                tbuf[s].at[:, :, :, pl.ds(0, 128)],
                out_hbm.at[pl.ds(g * KU, KU), :, wid],
                ssem[s],
            )

        def store_wait(g, s):
            pltpu.make_async_copy(
                tbuf[s].at[:, :, :, pl.ds(0, 128)],
                out_hbm.at[pl.ds(g * KU, KU), :, wid],
                ssem[s],
            ).wait()

        esub = lanes & 7
        band0 = lanes >> 3
        band1 = band0 + 2

        def transpose_relu(s):
            src = gbuf[s]
            dst = tbuf[s]
            for u in range(KU):
                ub = jnp.full((16,), u, jnp.int32)

                @pl.loop(0, 128, unroll=2)
                def _row(r):
                    rb = jnp.full((16,), r, jnp.int32)
                    row = u * 128 + r
                    v0 = jnp.maximum(src[row, 0:16], 0.0)
                    plsc.store_scatter(dst, [ub, band0, esub, rb], v0)
                    v1 = jnp.maximum(src[row, 16:32], 0.0)
                    plsc.store_scatter(dst, [ub, band1, esub, rb], v1)

        gather_start(0, 0)

        @pl.loop(0, NGRP, step=2)
        def _pair(G):
            for s in range(2):
                g = G + s
                o = 1 - s

                @pl.when(g >= 1)
                def _drain_prev_store():
                    store_wait(g - 1, o)

                @pl.when(g + 1 < NGRP)
                def _fire_next_gather():
                    gather_start(g + 1, o)

                gather_wait(g, s)
                transpose_relu(s)
                store_start(g, s)

        store_wait(NGRP - 1, 1)

    return emb_kernel


_FMT_KERNEL = _make_formatter()
_EMB_KERNEL = _make_kernel()


@jax.jit
def kernel(x, table):
    x1 = _FMT_KERNEL(x.astype(jnp.int32).T)
    out5 = _EMB_KERNEL(table, x1)
    return out5.transpose(2, 4, 0, 1, 3).reshape(B, L, EMBD)


# speedup vs baseline: 1.0189x; 1.0189x over previous
import functools

import jax
import jax.numpy as jnp
from jax import lax
from jax.experimental import pallas as pl
from jax.experimental.pallas import tpu as pltpu
from jax.experimental.pallas import tpu_sc as plsc

VOCAB = 1000000
EMBD = 32
B = 4096
L = 200

NC = 2
NS = 16
NW = NC * NS

BT = B // 128
LB = L // 8
TPAD = 131
KU = 5
NGRP = L // KU


def _make_formatter():
    mesh = plsc.VectorSubcoreMesh(core_axis_name="c", subcore_axis_name="s")

    @functools.partial(
        pl.kernel,
        out_type=jax.ShapeDtypeStruct((B * L,), jnp.int32),
        mesh=mesh,
        compiler_params=pltpu.CompilerParams(use_tc_tiling_on_sc=True),
        scratch_types=[
            pltpu.VMEM((LB, 8, 128), jnp.int32),
            pltpu.VMEM((LB * 1024,), jnp.int32),
            pltpu.SemaphoreType.DMA,
        ],
    )
    def fmt_kernel(xt_hbm, out_hbm, buf, flat, sem):
        wid = lax.axis_index("s") * NC + lax.axis_index("c")
        for lb in range(LB):
            pltpu.async_copy(
                xt_hbm.at[pl.ds(lb * 8, 8), pl.ds(wid * 128, 128)],
                buf.at[lb],
                sem,
            )
        for lb in range(LB):
            pltpu.make_async_copy(
                xt_hbm.at[pl.ds(lb * 8, 8), pl.ds(wid * 128, 128)],
                buf.at[lb],
                sem,
            ).wait()

        @pl.loop(0, LB)
        def _flatten(lb):
            for r in range(8):
                for c in range(8):
                    flat[pl.ds(lb * 1024 + r * 128 + c * 16, 16)] = (
                        buf[lb, r, pl.ds(c * 16, 16)]
                    )

        @pl.loop(0, LB)
        def _out(lb):
            pltpu.sync_copy(
                flat.at[pl.ds(lb * 1024, 1024)],
                out_hbm.at[pl.ds((lb * BT + wid) * 1024, 1024)],
            )

    return fmt_kernel


def _make_kernel():
    mesh = plsc.VectorSubcoreMesh(core_axis_name="c", subcore_axis_name="s")

    @functools.partial(
        pl.kernel,
        out_type=jax.ShapeDtypeStruct((L, EMBD // 8, BT, 8, 128), jnp.float32),
        mesh=mesh,
        compiler_params=pltpu.CompilerParams(
            use_tc_tiling_on_sc=False, needs_layout_passes=False
        ),
        scratch_types=[
            pltpu.VMEM((LB, 1024), jnp.int32),
            pltpu.VMEM((KU * 128, EMBD), jnp.float32),
            pltpu.VMEM((KU * 128, EMBD), jnp.float32),
            pltpu.VMEM((KU, EMBD // 8, 8, TPAD), jnp.float32),
            pltpu.VMEM((KU, EMBD // 8, 8, TPAD), jnp.float32),
            pltpu.SemaphoreType.DMA,
            pltpu.SemaphoreType.DMA,
            pltpu.SemaphoreType.DMA,
            pltpu.SemaphoreType.DMA,
        ],
    )
    def emb_kernel(table_hbm, x1_hbm, out_hbm,
                   idx_v, gb0, gb1, tb0, tb1, g0, g1, s0, s1):
        gbuf = (gb0, gb1)
        tbuf = (tb0, tb1)
        gsem = (g0, g1)
        ssem = (s0, s1)
        wid = lax.axis_index("s") * NC + lax.axis_index("c")
        lanes = lax.iota(jnp.int32, 16)

        for lb in range(LB):
            pltpu.async_copy(
                x1_hbm.at[pl.ds((lb * BT + wid) * 1024, 1024)],
                idx_v.at[lb],
                g0,
            )
        for lb in range(LB):
            pltpu.make_async_copy(
                x1_hbm.at[pl.ds((lb * BT + wid) * 1024, 1024)],
                idx_v.at[lb],
                g0,
            ).wait()

        def gather_start(g, s):
            for u in range(KU):
                l = g * KU + u
                pltpu.async_copy(
                    table_hbm.at[idx_v.at[l // 8, pl.ds((l % 8) * 128, 128)]],
                    gbuf[s].at[pl.ds(u * 128, 128)],
                    gsem[s],
                )

        def gather_wait(g, s):
            for u in range(KU):
                l = g * KU + u
                pltpu.make_async_copy(
                    table_hbm.at[idx_v.at[l // 8, pl.ds((l % 8) * 128, 128)]],
                    gbuf[s].at[pl.ds(u * 128, 128)],
                    gsem[s],
                ).wait()

        def store_start(g, s):
            pltpu.async_copy(
                tbuf[s].at[:, :, :, pl.ds(0, 128)],
                out_hbm.at[pl.ds(g * KU, KU), :, wid],
                ssem[s],
            )

        def store_wait(g, s):
            pltpu.make_async_copy(
                tbuf[s].at[:, :, :, pl.ds(0, 128)],
                out_hbm.at[pl.ds(g * KU, KU), :, wid],
                ssem[s],
            ).wait()

        esub = lanes & 7
        band0 = lanes >> 3
        band1 = band0 + 2

        def transpose_relu(s):
            src = gbuf[s]
            dst = tbuf[s]
            for u in range(KU):
                ub = jnp.full((16,), u, jnp.int32)

                @pl.loop(0, 128, unroll=8)
                def _row(r):
                    rb = jnp.full((16,), r, jnp.int32)
                    row = u * 128 + r
                    v0 = jnp.maximum(src[row, 0:16], 0.0)
                    plsc.store_scatter(dst, [ub, band0, esub, rb], v0)
                    v1 = jnp.maximum(src[row, 16:32], 0.0)
                    plsc.store_scatter(dst, [ub, band1, esub, rb], v1)

        gather_start(0, 0)

        @pl.loop(0, NGRP, step=2)
        def _pair(G):
            for s in range(2):
                g = G + s
                o = 1 - s

                @pl.when(g >= 1)
                def _drain_prev_store():
                    store_wait(g - 1, o)

                @pl.when(g + 1 < NGRP)
                def _fire_next_gather():
                    gather_start(g + 1, o)

                gather_wait(g, s)
                transpose_relu(s)
                store_start(g, s)

        store_wait(NGRP - 1, 1)

    return emb_kernel


_FMT_KERNEL = _make_formatter()
_EMB_KERNEL = _make_kernel()


@jax.jit
def kernel(x, table):
    x1 = _FMT_KERNEL(x.astype(jnp.int32).T)
    out5 = _EMB_KERNEL(table, x1)
    return out5.transpose(2, 4, 0, 1, 3).reshape(B, L, EMBD)


# speedup vs baseline: 1.0213x; 1.0024x over previous
import functools

import jax
import jax.numpy as jnp
from jax import lax
from jax.experimental import pallas as pl
from jax.experimental.pallas import tpu as pltpu
from jax.experimental.pallas import tpu_sc as plsc

VOCAB = 1000000
EMBD = 32
B = 4096
L = 200

NC = 2
NS = 16
NW = NC * NS

BT = B // 128
LB = L // 8
TPAD = 131
KU = 5
NGRP = L // KU


def _make_formatter():
    mesh = plsc.VectorSubcoreMesh(core_axis_name="c", subcore_axis_name="s")

    @functools.partial(
        pl.kernel,
        out_type=jax.ShapeDtypeStruct((B * L,), jnp.int32),
        mesh=mesh,
        compiler_params=pltpu.CompilerParams(use_tc_tiling_on_sc=True),
        scratch_types=[
            pltpu.VMEM((LB, 8, 128), jnp.int32),
            pltpu.VMEM((LB * 1024,), jnp.int32),
            pltpu.SemaphoreType.DMA,
        ],
    )
    def fmt_kernel(xt_hbm, out_hbm, buf, flat, sem):
        wid = lax.axis_index("s") * NC + lax.axis_index("c")
        for lb in range(LB):
            pltpu.async_copy(
                xt_hbm.at[pl.ds(lb * 8, 8), pl.ds(wid * 128, 128)],
                buf.at[lb],
                sem,
            )
        for lb in range(LB):
            pltpu.make_async_copy(
                xt_hbm.at[pl.ds(lb * 8, 8), pl.ds(wid * 128, 128)],
                buf.at[lb],
                sem,
            ).wait()

        @pl.loop(0, LB)
        def _flatten(lb):
            for r in range(8):
                for c in range(8):
                    flat[pl.ds(lb * 1024 + r * 128 + c * 16, 16)] = (
                        buf[lb, r, pl.ds(c * 16, 16)]
                    )

        @pl.loop(0, LB)
        def _out(lb):
            pltpu.sync_copy(
                flat.at[pl.ds(lb * 1024, 1024)],
                out_hbm.at[pl.ds((lb * BT + wid) * 1024, 1024)],
            )

    return fmt_kernel


def _make_kernel():
    mesh = plsc.VectorSubcoreMesh(core_axis_name="c", subcore_axis_name="s")

    @functools.partial(
        pl.kernel,
        out_type=jax.ShapeDtypeStruct((L, EMBD // 8, BT, 8, 128), jnp.float32),
        mesh=mesh,
        compiler_params=pltpu.CompilerParams(
            use_tc_tiling_on_sc=False, needs_layout_passes=False
        ),
        scratch_types=[
            pltpu.VMEM((LB, 1024), jnp.int32),
            pltpu.VMEM((KU * 128, EMBD), jnp.float32),
            pltpu.VMEM((KU * 128, EMBD), jnp.float32),
            pltpu.VMEM((KU, EMBD // 8, 8, TPAD), jnp.float32),
            pltpu.VMEM((KU, EMBD // 8, 8, TPAD), jnp.float32),
            pltpu.SemaphoreType.DMA,
            pltpu.SemaphoreType.DMA,
            pltpu.SemaphoreType.DMA,
            pltpu.SemaphoreType.DMA,
        ],
    )
    def emb_kernel(table_hbm, x1_hbm, out_hbm,
                   idx_v, gb0, gb1, tb0, tb1, g0, g1, s0, s1):
        gbuf = (gb0, gb1)
        tbuf = (tb0, tb1)
        gsem = (g0, g1)
        ssem = (s0, s1)
        wid = lax.axis_index("s") * NC + lax.axis_index("c")
        lanes = lax.iota(jnp.int32, 16)

        for lb in range(LB):
            pltpu.async_copy(
                x1_hbm.at[pl.ds((lb * BT + wid) * 1024, 1024)],
                idx_v.at[lb],
                g0,
            )
        for lb in range(LB):
            pltpu.make_async_copy(
                x1_hbm.at[pl.ds((lb * BT + wid) * 1024, 1024)],
                idx_v.at[lb],
                g0,
            ).wait()

        def gather_start(g, s):
            for u in range(KU):
                l = g * KU + u
                pltpu.async_copy(
                    table_hbm.at[idx_v.at[l // 8, pl.ds((l % 8) * 128, 128)]],
                    gbuf[s].at[pl.ds(u * 128, 128)],
                    gsem[s],
                )

        def gather_wait(g, s):
            for u in range(KU):
                l = g * KU + u
                pltpu.make_async_copy(
                    table_hbm.at[idx_v.at[l // 8, pl.ds((l % 8) * 128, 128)]],
                    gbuf[s].at[pl.ds(u * 128, 128)],
                    gsem[s],
                ).wait()

        def store_start(g, s):
            pltpu.async_copy(
                tbuf[s].at[:, :, :, pl.ds(0, 128)],
                out_hbm.at[pl.ds(g * KU, KU), :, wid],
                ssem[s],
            )

        def store_wait(g, s):
            pltpu.make_async_copy(
                tbuf[s].at[:, :, :, pl.ds(0, 128)],
                out_hbm.at[pl.ds(g * KU, KU), :, wid],
                ssem[s],
            ).wait()

        esub = lanes & 7
        band0 = lanes >> 3
        band1 = band0 + 2

        def transpose_relu(s):
            src = gbuf[s]
            dst = tbuf[s]
            for u in range(KU):
                ub = jnp.full((16,), u, jnp.int32)

                @pl.loop(0, 128, unroll=8)
                def _row(r):
                    rb = jnp.full((16,), r, jnp.int32)
                    row = u * 128 + r
                    v0 = jnp.maximum(src[row, 0:16], 0.0)
                    plsc.store_scatter(dst, [ub, band0, esub, rb], v0)
                    v1 = jnp.maximum(src[row, 16:32], 0.0)
                    plsc.store_scatter(dst, [ub, band1, esub, rb], v1)

        gather_start(0, 0)

        @pl.loop(0, NGRP, step=2)
        def _pair(G):
            for s in range(2):
                g = G + s
                o = 1 - s

                gather_wait(g, s)

                @pl.when(g >= 1)
                def _drain_prev_store():
                    store_wait(g - 1, o)

                @pl.when(g + 1 < NGRP)
                def _fire_next_gather():
                    gather_start(g + 1, o)

                transpose_relu(s)
                store_start(g, s)

        store_wait(NGRP - 1, 1)

    return emb_kernel


_FMT_KERNEL = _make_formatter()
_EMB_KERNEL = _make_kernel()


@jax.jit
def kernel(x, table):
    x1 = _FMT_KERNEL(x.astype(jnp.int32).T)
    out5 = _EMB_KERNEL(table, x1)
    return out5.transpose(2, 4, 0, 1, 3).reshape(B, L, EMBD)
